# baseline (speedup 1.0000x reference)
import numpy as np

import jax
import jax.numpy as jnp
from jax import lax
from jax.experimental import pallas as pl
from jax.experimental.pallas import tpu as pltpu
from jax.experimental.pallas import tpu_sc as plsc

_N = 262144
_C = 128
_R = _N // _C
_D = 64
_LOGN = 18

_NC = 2
_NS = 16
_NW = _NC * _NS
_CH = 128


def _stage_table():
    rows = []
    for k in range(1, _LOGN + 1):
        j = 1 << (k - 1)
        while j >= 1:
            if j >= _C:
                bid = 7 + (j // _C).bit_length() - 1
            else:
                bid = j.bit_length() - 1
            da, ds = (0, k) if k <= 6 else (1, k - 7)
            rows.append((bid, da, ds))
            j >>= 1
    return np.asarray(rows, dtype=np.int32)


_STAGES = _stage_table()
_NSTAGES = len(_STAGES)


def _lex_gt(ap, ai, bp, bi):
    return (ap > bp) | ((ap == bp) & (ai > bi))


def _sort_body(params_ref, phi_ref, sorted_ref, idx_ref):
    r_io = lax.broadcasted_iota(jnp.int32, (_R, _C), 0)
    c_io = lax.broadcasted_iota(jnp.int32, (_R, _C), 1)

    def make_branch(axis, dist):
        dim = 1 - axis
        size = (_C, _R)[axis]
        pos = (c_io, r_io)[axis]
        ps = dist.bit_length() - 1
        is_b = ((pos >> ps) & 1) == 1
        pos_bit = (pos >> ps) & 1

        def branch(phi, idx, da, ds):
            fwd_p = pltpu.roll(phi, dist, dim)
            bwd_p = pltpu.roll(phi, size - dist, dim)
            fwd_i = pltpu.roll(idx, dist, dim)
            bwd_i = pltpu.roll(idx, size - dist, dim)
            pp = jnp.where(is_b, fwd_p, bwd_p)
            pi = jnp.where(is_b, fwd_i, bwd_i)
            dio = jnp.where(da == 0, c_io, r_io)
            take_c = ((pos_bit ^ (dio >> ds)) & 1) == 1
            gt = _lex_gt(phi, idx, pp, pi)
            take = gt ^ take_c
            return jnp.where(take, pp, phi), jnp.where(take, pi, idx)

        return branch

    branches = [make_branch(0, 1 << q) for q in range(7)]
    branches += [make_branch(1, 1 << p) for p in range(11)]

    def step(t, carry):
        phi, idx = carry
        bid = params_ref[t, 0]
        da = params_ref[t, 1]
        ds = params_ref[t, 2]
        return lax.switch(bid, branches, phi, idx, da, ds)

    phi0 = phi_ref[...]
    idx0 = r_io * _C + c_io
    phi, idx = lax.fori_loop(0, _NSTAGES, step, (phi0, idx0))
    sorted_ref[...] = phi
    idx_ref[...] = idx


def _sort(phi2):
    return pl.pallas_call(
        _sort_body,
        in_specs=[
            pl.BlockSpec(memory_space=pltpu.SMEM),
            pl.BlockSpec(memory_space=pltpu.VMEM),
        ],
        out_shape=[
            jax.ShapeDtypeStruct((_R, _C), jnp.float32),
            jax.ShapeDtypeStruct((_R, _C), jnp.int32),
        ],
    )(jnp.asarray(_STAGES), phi2)


def _gather_body(emb_hbm, idx_hbm, out_hbm, idx_v, rows_v, sem):
    wid = lax.axis_index("s") * _NC + lax.axis_index("c")
    n_chunks = _N // (_NW * _CH)
    pltpu.sync_copy(idx_hbm.at[pl.ds(wid * n_chunks, n_chunks)], idx_v)

    def step(q, carry):
        pltpu.async_copy(emb_hbm.at[idx_v.at[q]], rows_v, sem).wait()
        row0 = (wid * n_chunks + q) * _CH
        pltpu.sync_copy(rows_v, out_hbm.at[pl.ds(row0, _CH)])
        return carry

    lax.fori_loop(0, n_chunks, step, 0)


def _gather(emb, idx2):
    n_chunks = _N // (_NW * _CH)
    mesh = plsc.VectorSubcoreMesh(core_axis_name="c", subcore_axis_name="s")
    f = pl.kernel(
        _gather_body,
        out_type=jax.ShapeDtypeStruct((_N, _D), jnp.float32),
        mesh=mesh,
        compiler_params=pltpu.CompilerParams(use_tc_tiling_on_sc=False),
        scratch_types=[
            pltpu.VMEM((n_chunks, _CH), jnp.int32),
            pltpu.VMEM((_CH, _D), jnp.float32),
            pltpu.SemaphoreType.DMA,
        ],
    )
    return f(emb, idx2)


def kernel(key_phi, key_embed):
    phi2 = key_phi.reshape(_R, _C)
    sorted_phi, idx2 = _sort(phi2)
    emb = key_embed.reshape(_N, _D)
    out = _gather(emb, idx2)
    return (sorted_phi.reshape(1, _N), out.reshape(1, _N, _D))


# speedup vs baseline: 4.8622x; 4.8622x over previous
import numpy as np

import jax
import jax.numpy as jnp
from jax import lax
from jax.experimental import pallas as pl
from jax.experimental.pallas import tpu as pltpu
from jax.experimental.pallas import tpu_sc as plsc

_N = 262144
_C = 128
_R = _N // _C
_D = 64
_LOGN = 18

_NC = 2
_NS = 16
_NW = _NC * _NS
_CH = 128


def _lex_gt(ap, ai, bp, bi):
    return (ap > bp) | ((ap == bp) & (ai > bi))


def _sort_body(phit_ref, sorted_ref, idx_ref, dphi_ref, didx_ref):
    r_io = lax.broadcasted_iota(jnp.int32, (_R, _C), 0)
    c_io = lax.broadcasted_iota(jnp.int32, (_R, _C), 1)

    phi = jnp.transpose(phit_ref[...])
    idx = c_io * _R + r_io

    def row_stage(phi, idx, s, down):
        d = jnp.int32(1) << s
        dphi_ref[0:_R] = phi
        dphi_ref[_R:2 * _R] = phi
        didx_ref[0:_R] = idx
        didx_ref[_R:2 * _R] = idx
        up = dphi_ref[pl.ds(d, _R)]
        vp = dphi_ref[pl.ds(_R - d, _R)]
        ui = didx_ref[pl.ds(d, _R)]
        vi = didx_ref[pl.ds(_R - d, _R)]
        is_b = ((r_io >> s) & 1) == 1
        pp = jnp.where(is_b, vp, up)
        pi = jnp.where(is_b, vi, ui)
        gt = _lex_gt(phi, idx, pp, pi)
        take = gt ^ down ^ is_b
        return jnp.where(take, pp, phi), jnp.where(take, pi, idx)

    def phase1(k, carry):
        phi, idx = carry
        down = (jnp.where(k <= 10, (r_io >> k) & 1, c_io & 1)) == 1

        def st(t, c2):
            return row_stage(*c2, k - 1 - t, down)

        return lax.fori_loop(0, k, st, (phi, idx))

    phi, idx = lax.fori_loop(1, 12, phase1, (phi, idx))

    def phase2(k, carry):
        phi, idx = carry
        down = ((c_io >> (k - 11)) & 1) == 1
        for u in range(6, -1, -1):
            dist = 1 << u
            is_b = ((c_io >> u) & 1) == 1
            fwd_p = pltpu.roll(phi, dist, 1)
            bwd_p = pltpu.roll(phi, _C - dist, 1)
            fwd_i = pltpu.roll(idx, dist, 1)
            bwd_i = pltpu.roll(idx, _C - dist, 1)
            pp = jnp.where(is_b, fwd_p, bwd_p)
            pi = jnp.where(is_b, fwd_i, bwd_i)
            gt = _lex_gt(phi, idx, pp, pi)
            take = (gt ^ down ^ is_b) & (u <= k - 12)
            phi = jnp.where(take, pp, phi)
            idx = jnp.where(take, pi, idx)

        def st(t, c2):
            return row_stage(*c2, 10 - t, down)

        return lax.fori_loop(0, 11, st, (phi, idx))

    phi, idx = lax.fori_loop(12, _LOGN + 1, phase2, (phi, idx))
    sorted_ref[...] = jnp.transpose(phi)
    idx_ref[...] = jnp.transpose(idx)


def _sort(phi_t):
    return pl.pallas_call(
        _sort_body,
        out_shape=[
            jax.ShapeDtypeStruct((_C, _R), jnp.float32),
            jax.ShapeDtypeStruct((_C, _R), jnp.int32),
        ],
        scratch_shapes=[
            pltpu.VMEM((2 * _R, _C), jnp.float32),
            pltpu.VMEM((2 * _R, _C), jnp.int32),
        ],
    )(phi_t)


def _gather_body(emb_hbm, idx_hbm, out_hbm, idx_v, rows_v, sem):
    wid = lax.axis_index("s") * _NC + lax.axis_index("c")
    n_chunks = _N // (_NW * _CH)
    pltpu.sync_copy(idx_hbm.at[pl.ds(wid * n_chunks, n_chunks)], idx_v)

    def step(q, carry):
        pltpu.async_copy(emb_hbm.at[idx_v.at[q]], rows_v, sem).wait()
        row0 = (wid * n_chunks + q) * _CH
        pltpu.sync_copy(rows_v, out_hbm.at[pl.ds(row0, _CH)])
        return carry

    lax.fori_loop(0, n_chunks, step, 0)


def _gather(emb, idx2):
    n_chunks = _N // (_NW * _CH)
    mesh = plsc.VectorSubcoreMesh(core_axis_name="c", subcore_axis_name="s")
    f = pl.kernel(
        _gather_body,
        out_type=jax.ShapeDtypeStruct((_N, _D), jnp.float32),
        mesh=mesh,
        compiler_params=pltpu.CompilerParams(use_tc_tiling_on_sc=False),
        scratch_types=[
            pltpu.VMEM((n_chunks, _CH), jnp.int32),
            pltpu.VMEM((_CH, _D), jnp.float32),
            pltpu.SemaphoreType.DMA,
        ],
    )
    return f(emb, idx2)


def kernel(key_phi, key_embed):
    phi_t = key_phi.reshape(_C, _R)
    sorted_t, idx_t = _sort(phi_t)
    idx2 = idx_t.reshape(_N).reshape(_R, _C)
    emb = key_embed.reshape(_N, _D)
    out = _gather(emb, idx2)
    return (sorted_t.reshape(1, _N), out.reshape(1, _N, _D))


# speedup vs baseline: 4.9940x; 1.0271x over previous
import numpy as np

import jax
import jax.numpy as jnp
from jax import lax
from jax.experimental import pallas as pl
from jax.experimental.pallas import tpu as pltpu
from jax.experimental.pallas import tpu_sc as plsc

_N = 262144
_C = 128
_R = _N // _C
_D = 64
_LOGN = 18

_NC = 2
_NS = 16
_NW = _NC * _NS
_CH = 128


def _lex_gt(ap, ai, bp, bi):
    return (ap > bp) | ((ap == bp) & (ai > bi))


def _sort_body(phit_ref, sorted_ref, idx_ref, dphi_ref, didx_ref):
    r_io = lax.broadcasted_iota(jnp.int32, (_R, _C), 0)
    c_io = lax.broadcasted_iota(jnp.int32, (_R, _C), 1)

    phi = jnp.transpose(phit_ref[...])
    idx = c_io * _R + r_io

    _B = 1024

    def row_stage(phi, idx, s, down):
        d = jnp.int32(1) << s
        dphi_ref[_B:_B + _R] = phi
        didx_ref[_B:_B + _R] = idx
        up = dphi_ref[pl.ds(_B + d, _R)]
        vp = dphi_ref[pl.ds(_B - d, _R)]
        ui = didx_ref[pl.ds(_B + d, _R)]
        vi = didx_ref[pl.ds(_B - d, _R)]
        is_b = ((r_io >> s) & 1) == 1
        pp = jnp.where(is_b, vp, up)
        pi = jnp.where(is_b, vi, ui)
        gt = _lex_gt(phi, idx, pp, pi)
        take = gt ^ down ^ is_b
        return jnp.where(take, pp, phi), jnp.where(take, pi, idx)

    def phase1(k, carry):
        phi, idx = carry
        down = (jnp.where(k <= 10, (r_io >> k) & 1, c_io & 1)) == 1

        def st(t, c2):
            return row_stage(*c2, k - 1 - t, down)

        return lax.fori_loop(0, k, st, (phi, idx))

    phi, idx = lax.fori_loop(1, 12, phase1, (phi, idx))

    def phase2(k, carry):
        phi, idx = carry
        down = ((c_io >> (k - 11)) & 1) == 1
        for u in range(6, -1, -1):
            dist = 1 << u
            is_b = ((c_io >> u) & 1) == 1
            fwd_p = pltpu.roll(phi, dist, 1)
            bwd_p = pltpu.roll(phi, _C - dist, 1)
            fwd_i = pltpu.roll(idx, dist, 1)
            bwd_i = pltpu.roll(idx, _C - dist, 1)
            pp = jnp.where(is_b, fwd_p, bwd_p)
            pi = jnp.where(is_b, fwd_i, bwd_i)
            gt = _lex_gt(phi, idx, pp, pi)
            take = (gt ^ down ^ is_b) & (u <= k - 12)
            phi = jnp.where(take, pp, phi)
            idx = jnp.where(take, pi, idx)

        def st(t, c2):
            return row_stage(*c2, 10 - t, down)

        return lax.fori_loop(0, 11, st, (phi, idx))

    phi, idx = lax.fori_loop(12, _LOGN + 1, phase2, (phi, idx))
    sorted_ref[...] = jnp.transpose(phi)
    idx_ref[...] = jnp.transpose(idx)


def _sort(phi_t):
    return pl.pallas_call(
        _sort_body,
        out_shape=[
            jax.ShapeDtypeStruct((_C, _R), jnp.float32),
            jax.ShapeDtypeStruct((_C, _R), jnp.int32),
        ],
        scratch_shapes=[
            pltpu.VMEM((2 * _R, _C), jnp.float32),
            pltpu.VMEM((2 * _R, _C), jnp.int32),
        ],
    )(phi_t)


def _gather_body(emb_hbm, idx_hbm, out_hbm, idx_v, rows_v, sem):
    wid = lax.axis_index("s") * _NC + lax.axis_index("c")
    n_chunks = _N // (_NW * _CH)
    pltpu.sync_copy(idx_hbm.at[pl.ds(wid * n_chunks, n_chunks)], idx_v)

    def step(q, carry):
        pltpu.async_copy(emb_hbm.at[idx_v.at[q]], rows_v, sem).wait()
        row0 = (wid * n_chunks + q) * _CH
        pltpu.sync_copy(rows_v, out_hbm.at[pl.ds(row0, _CH)])
        return carry

    lax.fori_loop(0, n_chunks, step, 0)


def _gather(emb, idx2):
    n_chunks = _N // (_NW * _CH)
    mesh = plsc.VectorSubcoreMesh(core_axis_name="c", subcore_axis_name="s")
    f = pl.kernel(
        _gather_body,
        out_type=jax.ShapeDtypeStruct((_N, _D), jnp.float32),
        mesh=mesh,
        compiler_params=pltpu.CompilerParams(use_tc_tiling_on_sc=False),
        scratch_types=[
            pltpu.VMEM((n_chunks, _CH), jnp.int32),
            pltpu.VMEM((_CH, _D), jnp.float32),
            pltpu.SemaphoreType.DMA,
        ],
    )
    return f(emb, idx2)


def kernel(key_phi, key_embed):
    phi_t = key_phi.reshape(_C, _R)
    sorted_t, idx_t = _sort(phi_t)
    idx2 = idx_t.reshape(_N).reshape(_R, _C)
    emb = key_embed.reshape(_N, _D)
    out = _gather(emb, idx2)
    return (sorted_t.reshape(1, _N), out.reshape(1, _N, _D))
